# Initial kernel scaffold; baseline (speedup 1.0000x reference)
import jax
import jax.numpy as jnp
from jax.experimental import pallas as pl


def kernel(input_spikes, delays, spike_buffer):
    raise NotImplementedError("write your pallas kernel here")


# speedup vs baseline: 75.2115x; 75.2115x over previous
import functools

import jax
import jax.numpy as jnp
import numpy as np
from jax import lax
from jax.experimental import pallas as pl
from jax.experimental.pallas import tpu as pltpu
from jax.experimental.pallas import tpu_sc as plsc

S = 512
T = 512
BATCH = 256
BUF_LEN = 21
DT = np.float32(0.001)

_info = plsc.get_sparse_core_info()
NC = _info.num_cores
NS = _info.num_subcores
LANES = _info.num_lanes
NW = NC * NS
TPW = T // NW


def _sc_body(spikes_hbm, delays_hbm, sbuf_hbm, out_hbm, dcol, buf, obuf):
    c = lax.axis_index("c")
    s_ax = lax.axis_index("s")
    w = s_ax * NC + c
    t0 = w * TPW

    pltpu.sync_copy(delays_hbm.at[:, pl.ds(t0, TPW)], dcol)
    pltpu.sync_copy(sbuf_hbm, buf)
    pltpu.sync_copy(spikes_hbm.at[pl.ds(0, 1), :], buf.at[pl.ds(0, 1), :])

    def step(i, acc):
        dv = dcol[i]
        steps = (dv / DT).astype(jnp.int32)
        idx = jnp.where(steps == 0, 0, BUF_LEN - steps)
        idx = jnp.minimum(jnp.maximum(idx, 0), BUF_LEN - 1)
        sv = jnp.full((LANES,), i, jnp.int32)
        g = plsc.load_gather(buf, [idx, sv])
        return acc + jnp.where(steps < BUF_LEN, g, np.float32(0.0))

    acc = lax.fori_loop(0, S, step, jnp.zeros((LANES,), jnp.float32))

    for r in range(BATCH):
        obuf[r] = acc
    pltpu.sync_copy(obuf, out_hbm.at[:, pl.ds(t0, TPW)])


@jax.jit
def _run(input_spikes, delays, spike_buffer):
    mesh = plsc.VectorSubcoreMesh(core_axis_name="c", subcore_axis_name="s")
    return pl.kernel(
        _sc_body,
        out_type=jax.ShapeDtypeStruct((BATCH, T), jnp.float32),
        mesh=mesh,
        scratch_types=[
            pltpu.VMEM((S, TPW), jnp.float32),
            pltpu.VMEM((BUF_LEN, S), jnp.float32),
            pltpu.VMEM((BATCH, TPW), jnp.float32),
        ],
        compiler_params=pltpu.CompilerParams(
            use_tc_tiling_on_sc=False, needs_layout_passes=False
        ),
    )(input_spikes, delays, spike_buffer)


def kernel(input_spikes, delays, spike_buffer):
    return _run(input_spikes, delays, spike_buffer)
